# Initial kernel scaffold; baseline (speedup 1.0000x reference)
import jax
import jax.numpy as jnp
from jax.experimental import pallas as pl


def kernel(node_feats, edge_index):
    raise NotImplementedError("write your pallas kernel here")


# speedup vs baseline: 1.3659x; 1.3659x over previous
import functools

import jax
import jax.numpy as jnp
from jax import lax
from jax.experimental import pallas as pl
from jax.experimental.pallas import tpu as pltpu
from jax.experimental.pallas import tpu_sc as plsc

N_NODES = 10000
N_EDGES = 320000
D_FEAT = 128

NC = 2
NS = 16
NW = NC * NS
RANGE = 320
TRASH = RANGE
CHUNK = 2000
NCHUNKS = N_EDGES // CHUNK
NBLOCKS_OUT = RANGE // 16

_NEG_INF = float("-inf")


def _seg_max_kernel(feats_hbm, src_hbm, dst_hbm, out_hbm,
                    dst_v, src_v, src_sel, ldst_sel, rows_v, acc, sem):
    wid = lax.axis_index("s") * NC + lax.axis_index("c")
    lo = wid * RANGE
    hi = lo + RANGE

    neg = jnp.full((16,), _NEG_INF, jnp.float32)

    def init_body(r, _):
        for c in range(D_FEAT // 16):
            acc[r, pl.ds(c * 16, 16)] = neg
        return 0

    lax.fori_loop(0, RANGE + 1, init_body, 0)

    dummy_src = jnp.zeros((16,), jnp.int32)
    dummy_ldst = jnp.full((16,), TRASH, jnp.int32)

    def chunk_body(ch, _):
        base = ch * CHUNK
        pltpu.sync_copy(dst_hbm.at[pl.ds(base, CHUNK)], dst_v)
        pltpu.sync_copy(src_hbm.at[pl.ds(base, CHUNK)], src_v)

        def scan_body(i, k):
            d = dst_v[pl.ds(i * 16, 16)]
            m = (d >= lo) & (d < hi)
            s = src_v[pl.ds(i * 16, 16)]
            cs = plsc.cumsum(jnp.where(m, 1, 0))
            pos = (k - 1) + cs
            plsc.store_scatter(src_sel, [pos], s, mask=m)
            plsc.store_scatter(ldst_sel, [pos], d - lo, mask=m)
            return k + jnp.max(cs)

        k = lax.fori_loop(0, CHUNK // 16, scan_body, jnp.int32(0))

        pad_pos = k + lax.iota(jnp.int32, 16)
        plsc.store_scatter(src_sel, [pad_pos], dummy_src)
        plsc.store_scatter(ldst_sel, [pad_pos], dummy_ldst)
        nblk = (k + 15) // 16

        def drain_body(j, _):
            iv = src_sel[pl.ds(j * 16, 16)]
            pltpu.async_copy(feats_hbm.at[iv], rows_v, sem).wait()
            ldv = ldst_sel[pl.ds(j * 16, 16)]
            for r in range(16):
                ld = ldv[r]
                for c in range(D_FEAT // 16):
                    sl = pl.ds(c * 16, 16)
                    acc[ld, sl] = jnp.maximum(acc[ld, sl], rows_v[r, sl])
            return 0

        lax.fori_loop(0, nblk, drain_body, 0)
        return 0

    lax.fori_loop(0, NCHUNKS, chunk_body, 0)

    def fix_body(r, _):
        for c in range(D_FEAT // 16):
            sl = pl.ds(c * 16, 16)
            v = acc[r, sl]
            acc[r, sl] = jnp.where(v == _NEG_INF, 0.0, v)
        return 0

    lax.fori_loop(0, RANGE, fix_body, 0)

    def out_body(b, _):
        @pl.when(lo + b * 16 < N_NODES)
        def _():
            pltpu.sync_copy(acc.at[pl.ds(b * 16, 16)],
                            out_hbm.at[pl.ds(lo + b * 16, 16)])
        return 0

    lax.fori_loop(0, NBLOCKS_OUT, out_body, 0)


@jax.jit
def _seg_max(node_feats, src, dst):
    mesh = plsc.VectorSubcoreMesh(core_axis_name="c", subcore_axis_name="s")
    f = functools.partial(
        pl.kernel,
        mesh=mesh,
        out_type=jax.ShapeDtypeStruct((N_NODES, D_FEAT), jnp.float32),
        scratch_types=[
            pltpu.VMEM((CHUNK,), jnp.int32),
            pltpu.VMEM((CHUNK,), jnp.int32),
            pltpu.VMEM((CHUNK + 16,), jnp.int32),
            pltpu.VMEM((CHUNK + 16,), jnp.int32),
            pltpu.VMEM((16, D_FEAT), jnp.float32),
            pltpu.VMEM((RANGE + 1, D_FEAT), jnp.float32),
            pltpu.SemaphoreType.DMA,
        ],
        compiler_params=pltpu.CompilerParams(needs_layout_passes=False),
    )(_seg_max_kernel)
    return f(node_feats, src, dst)


def kernel(node_feats, edge_index):
    ei = edge_index.astype(jnp.int32)
    return _seg_max(node_feats, ei[0], ei[1])


# speedup vs baseline: 1.3874x; 1.0157x over previous
import functools

import jax
import jax.numpy as jnp
from jax import lax
from jax.experimental import pallas as pl
from jax.experimental.pallas import tpu as pltpu
from jax.experimental.pallas import tpu_sc as plsc

N_NODES = 10000
N_EDGES = 320000
D_FEAT = 128

NC = 2
NS = 16
NW = NC * NS
RANGE = 320
TRASH = RANGE
CHUNK = 2000
NCHUNKS = N_EDGES // CHUNK
NBLOCKS_OUT = RANGE // 16

_NEG_INF = float("-inf")


def _seg_max_kernel(feats_hbm, src_hbm, dst_hbm, out_hbm,
                    dst_v, src_v, src_sel, ldst_sel, rows0, rows1, acc,
                    sem0, sem1):
    wid = lax.axis_index("s") * NC + lax.axis_index("c")
    lo = wid * RANGE
    hi = lo + RANGE

    neg = jnp.full((16,), _NEG_INF, jnp.float32)

    def init_body(r, _):
        for c in range(D_FEAT // 16):
            acc[r, pl.ds(c * 16, 16)] = neg
        return 0

    lax.fori_loop(0, RANGE + 1, init_body, 0)

    dummy_src = jnp.zeros((16,), jnp.int32)
    dummy_ldst = jnp.full((16,), TRASH, jnp.int32)

    def chunk_body(ch, _):
        base = ch * CHUNK
        pltpu.sync_copy(dst_hbm.at[pl.ds(base, CHUNK)], dst_v)
        pltpu.sync_copy(src_hbm.at[pl.ds(base, CHUNK)], src_v)

        def scan_body(i, k):
            d = dst_v[pl.ds(i * 16, 16)]
            m = (d >= lo) & (d < hi)

            def compact(kk):
                s = src_v[pl.ds(i * 16, 16)]
                cs = plsc.cumsum(jnp.where(m, 1, 0))
                pos = (kk - 1) + cs
                plsc.store_scatter(src_sel, [pos], s, mask=m)
                plsc.store_scatter(ldst_sel, [pos], d - lo, mask=m)
                return kk + cs[15]

            return lax.cond(jnp.any(m), compact, lambda kk: kk, k)

        k = lax.fori_loop(0, CHUNK // 16, scan_body, jnp.int32(0))

        pad_pos = k + lax.iota(jnp.int32, 16)
        plsc.store_scatter(src_sel, [pad_pos], dummy_src)
        plsc.store_scatter(ldst_sel, [pad_pos], dummy_ldst)
        nblk = (k + 15) // 16

        def fire(j, buf, sem):
            iv = src_sel[pl.ds(j * 16, 16)]
            pltpu.async_copy(feats_hbm.at[iv], buf, sem)

        def wait(buf, sem):
            pltpu.make_async_copy(feats_hbm.at[pl.ds(0, 16)], buf, sem).wait()

        def reduce_block(j, buf):
            ldv = ldst_sel[pl.ds(j * 16, 16)]
            for r in range(16):
                ld = ldv[r]
                for c in range(D_FEAT // 16):
                    sl = pl.ds(c * 16, 16)
                    acc[ld, sl] = jnp.maximum(acc[ld, sl], buf[r, sl])

        @pl.when(nblk > 0)
        def _():
            fire(0, rows0, sem0)

        @pl.when(nblk > 1)
        def _():
            fire(1, rows1, sem1)

        def drain2(jj, _):
            j0 = 2 * jj

            @pl.when(j0 < nblk)
            def _():
                wait(rows0, sem0)
                reduce_block(j0, rows0)

                @pl.when(j0 + 2 < nblk)
                def _():
                    fire(j0 + 2, rows0, sem0)

            j1 = j0 + 1

            @pl.when(j1 < nblk)
            def _():
                wait(rows1, sem1)
                reduce_block(j1, rows1)

                @pl.when(j1 + 2 < nblk)
                def _():
                    fire(j1 + 2, rows1, sem1)

            return 0

        lax.fori_loop(0, (nblk + 1) // 2, drain2, 0)
        return 0

    lax.fori_loop(0, NCHUNKS, chunk_body, 0)

    def fix_body(r, _):
        for c in range(D_FEAT // 16):
            sl = pl.ds(c * 16, 16)
            v = acc[r, sl]
            acc[r, sl] = jnp.where(v == _NEG_INF, 0.0, v)
        return 0

    lax.fori_loop(0, RANGE, fix_body, 0)

    def out_body(b, _):
        @pl.when(lo + b * 16 < N_NODES)
        def _():
            pltpu.sync_copy(acc.at[pl.ds(b * 16, 16)],
                            out_hbm.at[pl.ds(lo + b * 16, 16)])
        return 0

    lax.fori_loop(0, NBLOCKS_OUT, out_body, 0)


@jax.jit
def _seg_max(node_feats, src, dst):
    mesh = plsc.VectorSubcoreMesh(core_axis_name="c", subcore_axis_name="s")
    f = functools.partial(
        pl.kernel,
        mesh=mesh,
        out_type=jax.ShapeDtypeStruct((N_NODES, D_FEAT), jnp.float32),
        scratch_types=[
            pltpu.VMEM((CHUNK,), jnp.int32),
            pltpu.VMEM((CHUNK,), jnp.int32),
            pltpu.VMEM((CHUNK + 16,), jnp.int32),
            pltpu.VMEM((CHUNK + 16,), jnp.int32),
            pltpu.VMEM((16, D_FEAT), jnp.float32),
            pltpu.VMEM((16, D_FEAT), jnp.float32),
            pltpu.VMEM((RANGE + 1, D_FEAT), jnp.float32),
            pltpu.SemaphoreType.DMA,
            pltpu.SemaphoreType.DMA,
        ],
        compiler_params=pltpu.CompilerParams(needs_layout_passes=False),
    )(_seg_max_kernel)
    return f(node_feats, src, dst)


def kernel(node_feats, edge_index):
    ei = edge_index.astype(jnp.int32)
    return _seg_max(node_feats, ei[0], ei[1])


# speedup vs baseline: 2.4215x; 1.7454x over previous
import functools

import jax
import jax.numpy as jnp
from jax import lax
from jax.experimental import pallas as pl
from jax.experimental.pallas import tpu as pltpu
from jax.experimental.pallas import tpu_sc as plsc

N_NODES = 10000
N_EDGES = 320000
D_FEAT = 128

NC = 2
NS = 16
NW = NC * NS
RANGE = 320
TRASH = RANGE
CHUNK = 4000
NCHUNKS = N_EDGES // CHUNK
NBLOCKS_OUT = RANGE // 16
UNROLL = 5
NRING = 8

_NEG_INF = float("-inf")


def _seg_max_kernel(feats_hbm, src_hbm, dst_hbm, out_hbm,
                    dst_v, src_v, src_sel, ldst_sel, ring, acc, sem):
    wid = lax.axis_index("s") * NC + lax.axis_index("c")
    lo = wid * RANGE
    hi = lo + RANGE

    neg = jnp.full((16,), _NEG_INF, jnp.float32)

    def init_body(r, _):
        for c in range(D_FEAT // 16):
            acc[r, pl.ds(c * 16, 16)] = neg
        return 0

    lax.fori_loop(0, RANGE + 1, init_body, 0)

    dummy_src = jnp.zeros((16,), jnp.int32)
    dummy_ldst = jnp.full((16,), TRASH, jnp.int32)

    def chunk_body(ch, _):
        base = ch * CHUNK
        pltpu.sync_copy(dst_hbm.at[pl.ds(base, CHUNK)], dst_v)
        pltpu.sync_copy(src_hbm.at[pl.ds(base, CHUNK)], src_v)

        def scan_body(i, k):
            for u in range(UNROLL):
                off = (i * UNROLL + u) * 16
                d = dst_v[pl.ds(off, 16)]
                m = (d >= lo) & (d < hi)
                s = src_v[pl.ds(off, 16)]
                cs = plsc.cumsum(jnp.where(m, 1, 0))
                pos = (k - 1) + cs
                plsc.store_scatter(src_sel, [pos], s, mask=m)
                plsc.store_scatter(ldst_sel, [pos], d - lo, mask=m)
                k = k + cs[15]
            return k

        k = lax.fori_loop(0, CHUNK // 16 // UNROLL, scan_body, jnp.int32(0))

        pad_pos = k + lax.iota(jnp.int32, 16)
        plsc.store_scatter(src_sel, [pad_pos], dummy_src)
        plsc.store_scatter(ldst_sel, [pad_pos], dummy_ldst)
        nblk = (k + 15) // 16

        def fire(j):
            iv = src_sel[pl.ds(j * 16, 16)]
            pltpu.async_copy(feats_hbm.at[iv], ring.at[j % NRING], sem)

        def prime(j, _):
            @pl.when(j < nblk)
            def _():
                fire(j)
            return 0

        lax.fori_loop(0, NRING, prime, 0)

        def drain_body(j, _):
            slot = j % NRING
            pltpu.make_async_copy(feats_hbm.at[pl.ds(0, 16)],
                                  ring.at[slot], sem).wait()
            ldv = ldst_sel[pl.ds(j * 16, 16)]
            for r in range(16):
                ld = ldv[r]
                for c in range(D_FEAT // 16):
                    sl = pl.ds(c * 16, 16)
                    acc[ld, sl] = jnp.maximum(acc[ld, sl], ring[slot, r, sl])

            @pl.when(j + NRING < nblk)
            def _():
                fire(j + NRING)

            return 0

        lax.fori_loop(0, nblk, drain_body, 0)
        return 0

    lax.fori_loop(0, NCHUNKS, chunk_body, 0)

    def fix_body(r, _):
        for c in range(D_FEAT // 16):
            sl = pl.ds(c * 16, 16)
            v = acc[r, sl]
            acc[r, sl] = jnp.where(v == _NEG_INF, 0.0, v)
        return 0

    lax.fori_loop(0, RANGE, fix_body, 0)

    def out_body(b, _):
        @pl.when(lo + b * 16 < N_NODES)
        def _():
            pltpu.sync_copy(acc.at[pl.ds(b * 16, 16)],
                            out_hbm.at[pl.ds(lo + b * 16, 16)])
        return 0

    lax.fori_loop(0, NBLOCKS_OUT, out_body, 0)


@jax.jit
def _seg_max(node_feats, src, dst):
    mesh = plsc.VectorSubcoreMesh(core_axis_name="c", subcore_axis_name="s")
    f = functools.partial(
        pl.kernel,
        mesh=mesh,
        out_type=jax.ShapeDtypeStruct((N_NODES, D_FEAT), jnp.float32),
        scratch_types=[
            pltpu.VMEM((CHUNK,), jnp.int32),
            pltpu.VMEM((CHUNK,), jnp.int32),
            pltpu.VMEM((CHUNK + 16,), jnp.int32),
            pltpu.VMEM((CHUNK + 16,), jnp.int32),
            pltpu.VMEM((NRING, 16, D_FEAT), jnp.float32),
            pltpu.VMEM((RANGE + 1, D_FEAT), jnp.float32),
            pltpu.SemaphoreType.DMA,
        ],
        compiler_params=pltpu.CompilerParams(needs_layout_passes=False),
    )(_seg_max_kernel)
    return f(node_feats, src, dst)


def kernel(node_feats, edge_index):
    ei = edge_index.astype(jnp.int32)
    return _seg_max(node_feats, ei[0], ei[1])


# speedup vs baseline: 2.6210x; 1.0824x over previous
import functools

import jax
import jax.numpy as jnp
from jax import lax
from jax.experimental import pallas as pl
from jax.experimental.pallas import tpu as pltpu
from jax.experimental.pallas import tpu_sc as plsc

N_NODES = 10000
N_EDGES = 320000
D_FEAT = 128

NC = 2
NS = 16
NW = NC * NS
RANGE = 320
TRASH = RANGE
CHUNK = 4000
NCHUNKS = N_EDGES // CHUNK
NBLOCKS_OUT = RANGE // 16
UNROLL = 5
NRING = 8

_NEG_INF = float("-inf")


def _seg_max_kernel(feats_hbm, src_hbm, dst_hbm, out_hbm,
                    dst0, src0, dst1, src1, src_sel, ldst_sel, ring, acc, sem,
                    sem_e0, sem_e1):
    wid = lax.axis_index("s") * NC + lax.axis_index("c")
    lo = wid * RANGE
    hi = lo + RANGE

    neg = jnp.full((16,), _NEG_INF, jnp.float32)

    def init_body(r, _):
        for c in range(D_FEAT // 16):
            acc[r, pl.ds(c * 16, 16)] = neg
        return 0

    lax.fori_loop(0, RANGE + 1, init_body, 0)

    dummy_src = jnp.zeros((16,), jnp.int32)
    dummy_ldst = jnp.full((16,), TRASH, jnp.int32)

    def fire_chunk(ch, dbuf, sbuf, sem_e):
        base = ch * CHUNK
        pltpu.async_copy(dst_hbm.at[pl.ds(base, CHUNK)], dbuf, sem_e)
        pltpu.async_copy(src_hbm.at[pl.ds(base, CHUNK)], sbuf, sem_e)

    def wait_chunk(dbuf, sbuf, sem_e):
        pltpu.make_async_copy(dst_hbm.at[pl.ds(0, CHUNK)], dbuf, sem_e).wait()
        pltpu.make_async_copy(src_hbm.at[pl.ds(0, CHUNK)], sbuf, sem_e).wait()

    def chunk_body(ch, dbuf, sbuf, sem_e):
        wait_chunk(dbuf, sbuf, sem_e)

        def scan_body(i, k):
            for u in range(UNROLL):
                off = (i * UNROLL + u) * 16
                d = dbuf[pl.ds(off, 16)]
                m = (d >= lo) & (d < hi)
                s = sbuf[pl.ds(off, 16)]
                cs = plsc.cumsum(jnp.where(m, 1, 0))
                pos = (k - 1) + cs
                plsc.store_scatter(src_sel, [pos], s, mask=m)
                plsc.store_scatter(ldst_sel, [pos], d - lo, mask=m)
                k = k + cs[15]
            return k

        k = lax.fori_loop(0, CHUNK // 16 // UNROLL, scan_body, jnp.int32(0))

        pad_pos = k + lax.iota(jnp.int32, 16)
        plsc.store_scatter(src_sel, [pad_pos], dummy_src)
        plsc.store_scatter(ldst_sel, [pad_pos], dummy_ldst)
        nblk = (k + 15) // 16

        def fire(j):
            iv = src_sel[pl.ds(j * 16, 16)]
            pltpu.async_copy(feats_hbm.at[iv], ring.at[j % NRING], sem)

        def prime(j, _):
            @pl.when(j < nblk)
            def _():
                fire(j)
            return 0

        lax.fori_loop(0, NRING, prime, 0)

        def drain_body(j, _):
            slot = j % NRING
            pltpu.make_async_copy(feats_hbm.at[pl.ds(0, 16)],
                                  ring.at[slot], sem).wait()
            ldv = ldst_sel[pl.ds(j * 16, 16)]
            lds = [ldv[r] for r in range(16)]
            for r in range(16):
                ld = lds[r]
                for c in range(D_FEAT // 16):
                    sl = pl.ds(c * 16, 16)
                    acc[ld, sl] = jnp.maximum(acc[ld, sl], ring[slot, r, sl])

            @pl.when(j + NRING < nblk)
            def _():
                fire(j + NRING)

            return 0

        lax.fori_loop(0, nblk, drain_body, 0)

    fire_chunk(0, dst0, src0, sem_e0)

    def outer_body(jj, _):
        ch0 = 2 * jj

        @pl.when(ch0 + 1 < NCHUNKS)
        def _():
            fire_chunk(ch0 + 1, dst1, src1, sem_e1)

        chunk_body(ch0, dst0, src0, sem_e0)

        @pl.when(ch0 + 2 < NCHUNKS)
        def _():
            fire_chunk(ch0 + 2, dst0, src0, sem_e0)

        @pl.when(ch0 + 1 < NCHUNKS)
        def _():
            chunk_body(ch0 + 1, dst1, src1, sem_e1)

        return 0

    lax.fori_loop(0, (NCHUNKS + 1) // 2, outer_body, 0)

    def fix_body(r, _):
        for c in range(D_FEAT // 16):
            sl = pl.ds(c * 16, 16)
            v = acc[r, sl]
            acc[r, sl] = jnp.where(v == _NEG_INF, 0.0, v)
        return 0

    lax.fori_loop(0, RANGE, fix_body, 0)

    def out_body(b, _):
        @pl.when(lo + b * 16 < N_NODES)
        def _():
            pltpu.sync_copy(acc.at[pl.ds(b * 16, 16)],
                            out_hbm.at[pl.ds(lo + b * 16, 16)])
        return 0

    lax.fori_loop(0, NBLOCKS_OUT, out_body, 0)


@jax.jit
def _seg_max(node_feats, src, dst):
    mesh = plsc.VectorSubcoreMesh(core_axis_name="c", subcore_axis_name="s")
    f = functools.partial(
        pl.kernel,
        mesh=mesh,
        out_type=jax.ShapeDtypeStruct((N_NODES, D_FEAT), jnp.float32),
        scratch_types=[
            pltpu.VMEM((CHUNK,), jnp.int32),
            pltpu.VMEM((CHUNK,), jnp.int32),
            pltpu.VMEM((CHUNK,), jnp.int32),
            pltpu.VMEM((CHUNK,), jnp.int32),
            pltpu.VMEM((CHUNK + 16,), jnp.int32),
            pltpu.VMEM((CHUNK + 16,), jnp.int32),
            pltpu.VMEM((NRING, 16, D_FEAT), jnp.float32),
            pltpu.VMEM((RANGE + 1, D_FEAT), jnp.float32),
            pltpu.SemaphoreType.DMA,
            pltpu.SemaphoreType.DMA,
            pltpu.SemaphoreType.DMA,
        ],
        compiler_params=pltpu.CompilerParams(needs_layout_passes=False),
    )(_seg_max_kernel)
    return f(node_feats, src, dst)


def kernel(node_feats, edge_index):
    ei = edge_index.astype(jnp.int32)
    return _seg_max(node_feats, ei[0], ei[1])


# speedup vs baseline: 2.6233x; 1.0009x over previous
import functools

import jax
import jax.numpy as jnp
from jax import lax
from jax.experimental import pallas as pl
from jax.experimental.pallas import tpu as pltpu
from jax.experimental.pallas import tpu_sc as plsc

N_NODES = 10000
N_EDGES = 320000
D_FEAT = 128

NC = 2
NS = 16
NW = NC * NS
RANGE = 320
TRASH = RANGE
CHUNK = 4000
NCHUNKS = N_EDGES // CHUNK
NBLOCKS_OUT = RANGE // 16
UNROLL = 5
NRING = 8

_NEG_INF = float("-inf")


def _seg_max_kernel(feats_hbm, src_hbm, dst_hbm, out_hbm,
                    dst0, src0, dst1, src1, sel, ring, acc,
                    sem, sem_e0, sem_e1):
    wid = lax.axis_index("s") * NC + lax.axis_index("c")
    lo = wid * RANGE
    hi = lo + RANGE

    neg = jnp.full((16,), _NEG_INF, jnp.float32)

    def init_body(r, _):
        for c in range(D_FEAT // 16):
            acc[r, pl.ds(c * 16, 16)] = neg
        return 0

    lax.fori_loop(0, RANGE + 1, init_body, 0)

    dummy_packed = jnp.full((16,), TRASH, jnp.int32)

    def fire_chunk(ch, dbuf, sbuf, sem_e):
        base = ch * CHUNK
        pltpu.async_copy(dst_hbm.at[pl.ds(base, CHUNK)], dbuf, sem_e)
        pltpu.async_copy(src_hbm.at[pl.ds(base, CHUNK)], sbuf, sem_e)

    def wait_chunk(dbuf, sbuf, sem_e):
        pltpu.make_async_copy(dst_hbm.at[pl.ds(0, CHUNK)], dbuf, sem_e).wait()
        pltpu.make_async_copy(src_hbm.at[pl.ds(0, CHUNK)], sbuf, sem_e).wait()

    def chunk_body(ch, dbuf, sbuf, sem_e):
        wait_chunk(dbuf, sbuf, sem_e)

        def scan_body(i, k):
            for u in range(UNROLL):
                off = (i * UNROLL + u) * 16
                d = dbuf[pl.ds(off, 16)]
                m = (d >= lo) & (d < hi)
                s = sbuf[pl.ds(off, 16)]
                packed = s * 512 + (d - lo)
                cs = plsc.cumsum(jnp.where(m, 1, 0))
                pos = (k - 1) + cs
                plsc.store_scatter(sel, [pos], packed, mask=m)
                k = k + cs[15]
            return k

        k = lax.fori_loop(0, CHUNK // 16 // UNROLL, scan_body, jnp.int32(0))

        pad_pos = k + lax.iota(jnp.int32, 16)
        plsc.store_scatter(sel, [pad_pos], dummy_packed)
        nblk = (k + 15) // 16

        def fire(j):
            iv = lax.shift_right_logical(sel[pl.ds(j * 16, 16)], 9)
            pltpu.async_copy(feats_hbm.at[iv], ring.at[j % NRING], sem)

        def prime(j, _):
            @pl.when(j < nblk)
            def _():
                fire(j)
            return 0

        lax.fori_loop(0, NRING, prime, 0)

        def drain_body(j, _):
            slot = j % NRING
            pltpu.make_async_copy(feats_hbm.at[pl.ds(0, 16)],
                                  ring.at[slot], sem).wait()
            ldv = sel[pl.ds(j * 16, 16)] & 511
            lds = [ldv[r] for r in range(16)]
            for r in range(16):
                ld = lds[r]
                for c in range(D_FEAT // 16):
                    sl = pl.ds(c * 16, 16)
                    acc[ld, sl] = jnp.maximum(acc[ld, sl], ring[slot, r, sl])

            @pl.when(j + NRING < nblk)
            def _():
                fire(j + NRING)

            return 0

        lax.fori_loop(0, nblk, drain_body, 0)

    fire_chunk(0, dst0, src0, sem_e0)

    def outer_body(jj, _):
        ch0 = 2 * jj

        @pl.when(ch0 + 1 < NCHUNKS)
        def _():
            fire_chunk(ch0 + 1, dst1, src1, sem_e1)

        chunk_body(ch0, dst0, src0, sem_e0)

        @pl.when(ch0 + 2 < NCHUNKS)
        def _():
            fire_chunk(ch0 + 2, dst0, src0, sem_e0)

        @pl.when(ch0 + 1 < NCHUNKS)
        def _():
            chunk_body(ch0 + 1, dst1, src1, sem_e1)

        return 0

    lax.fori_loop(0, (NCHUNKS + 1) // 2, outer_body, 0)

    def fix_body(r, _):
        for c in range(D_FEAT // 16):
            sl = pl.ds(c * 16, 16)
            v = acc[r, sl]
            acc[r, sl] = jnp.where(v == _NEG_INF, 0.0, v)
        return 0

    lax.fori_loop(0, RANGE, fix_body, 0)

    def out_body(b, _):
        @pl.when(lo + b * 16 < N_NODES)
        def _():
            pltpu.sync_copy(acc.at[pl.ds(b * 16, 16)],
                            out_hbm.at[pl.ds(lo + b * 16, 16)])
        return 0

    lax.fori_loop(0, NBLOCKS_OUT, out_body, 0)


@jax.jit
def _seg_max(node_feats, src, dst):
    mesh = plsc.VectorSubcoreMesh(core_axis_name="c", subcore_axis_name="s")
    f = functools.partial(
        pl.kernel,
        mesh=mesh,
        out_type=jax.ShapeDtypeStruct((N_NODES, D_FEAT), jnp.float32),
        scratch_types=[
            pltpu.VMEM((CHUNK,), jnp.int32),
            pltpu.VMEM((CHUNK,), jnp.int32),
            pltpu.VMEM((CHUNK,), jnp.int32),
            pltpu.VMEM((CHUNK,), jnp.int32),
            pltpu.VMEM((CHUNK + 16,), jnp.int32),
            pltpu.VMEM((NRING, 16, D_FEAT), jnp.float32),
            pltpu.VMEM((RANGE + 1, D_FEAT), jnp.float32),
            pltpu.SemaphoreType.DMA,
            pltpu.SemaphoreType.DMA,
            pltpu.SemaphoreType.DMA,
        ],
        compiler_params=pltpu.CompilerParams(needs_layout_passes=False),
    )(_seg_max_kernel)
    return f(node_feats, src, dst)


def kernel(node_feats, edge_index):
    ei = edge_index.astype(jnp.int32)
    return _seg_max(node_feats, ei[0], ei[1])


# speedup vs baseline: 2.6262x; 1.0011x over previous
import functools

import jax
import jax.numpy as jnp
from jax import lax
from jax.experimental import pallas as pl
from jax.experimental.pallas import tpu as pltpu
from jax.experimental.pallas import tpu_sc as plsc

N_NODES = 10000
N_EDGES = 320000
D_FEAT = 128

NC = 2
NS = 16
NW = NC * NS
RANGE = 320
TRASH = RANGE
CHUNK = 4000
NCHUNKS = N_EDGES // CHUNK
NBLOCKS_OUT = RANGE // 16
UNROLL = 5
NRING = 8

_NEG_INF = float("-inf")


def _seg_max_kernel(feats_hbm, src_hbm, dst_hbm, out_hbm,
                    dst0, src0, dst1, src1, sel, ring, acc,
                    sem, sem_e0, sem_e1):
    wid = lax.axis_index("s") * NC + lax.axis_index("c")
    lo = wid * RANGE
    hi = lo + RANGE

    neg = jnp.full((16,), _NEG_INF, jnp.float32)

    def init_body(r, _):
        for c in range(D_FEAT // 16):
            acc[r, pl.ds(c * 16, 16)] = neg
        return 0

    lax.fori_loop(0, RANGE + 1, init_body, 0)

    dummy_packed = jnp.full((16,), TRASH, jnp.int32)

    def fire_chunk(ch, dbuf, sbuf, sem_e):
        base = ch * CHUNK
        pltpu.async_copy(dst_hbm.at[pl.ds(base, CHUNK)], dbuf, sem_e)
        pltpu.async_copy(src_hbm.at[pl.ds(base, CHUNK)], sbuf, sem_e)

    def wait_chunk(dbuf, sbuf, sem_e):
        pltpu.make_async_copy(dst_hbm.at[pl.ds(0, CHUNK)], dbuf, sem_e).wait()
        pltpu.make_async_copy(src_hbm.at[pl.ds(0, CHUNK)], sbuf, sem_e).wait()

    def chunk_body(ch, dbuf, sbuf, sem_e):
        wait_chunk(dbuf, sbuf, sem_e)

        def scan_body(i, kv):
            for u in range(UNROLL):
                off = (i * UNROLL + u) * 16
                d = dbuf[pl.ds(off, 16)]
                m = (d >= lo) & (d < hi)
                s = sbuf[pl.ds(off, 16)]
                packed = s * 512 + (d - lo)
                cs = plsc.cumsum(jnp.where(m, 1, 0))
                pos = (kv - 1) + cs
                plsc.store_scatter(sel, [pos], packed, mask=m)
                kv = kv + plsc.all_reduce_population_count(m)
            return kv

        kv = lax.fori_loop(0, CHUNK // 16 // UNROLL, scan_body,
                           jnp.zeros((16,), jnp.int32))
        k = kv[0]

        pad_pos = k + lax.iota(jnp.int32, 16)
        plsc.store_scatter(sel, [pad_pos], dummy_packed)
        nblk = (k + 15) // 16

        def fire(j):
            iv = lax.shift_right_logical(sel[pl.ds(j * 16, 16)], 9)
            pltpu.async_copy(feats_hbm.at[iv], ring.at[j % NRING], sem)

        def prime(j, _):
            @pl.when(j < nblk)
            def _():
                fire(j)
            return 0

        lax.fori_loop(0, NRING, prime, 0)

        def drain_body(j, _):
            slot = j % NRING
            pltpu.make_async_copy(feats_hbm.at[pl.ds(0, 16)],
                                  ring.at[slot], sem).wait()
            ldv = sel[pl.ds(j * 16, 16)] & 511
            lds = [ldv[r] for r in range(16)]
            for r in range(16):
                ld = lds[r]
                for c in range(D_FEAT // 16):
                    sl = pl.ds(c * 16, 16)
                    acc[ld, sl] = jnp.maximum(acc[ld, sl], ring[slot, r, sl])

            @pl.when(j + NRING < nblk)
            def _():
                fire(j + NRING)

            return 0

        lax.fori_loop(0, nblk, drain_body, 0)

    fire_chunk(0, dst0, src0, sem_e0)

    def outer_body(jj, _):
        ch0 = 2 * jj

        @pl.when(ch0 + 1 < NCHUNKS)
        def _():
            fire_chunk(ch0 + 1, dst1, src1, sem_e1)

        chunk_body(ch0, dst0, src0, sem_e0)

        @pl.when(ch0 + 2 < NCHUNKS)
        def _():
            fire_chunk(ch0 + 2, dst0, src0, sem_e0)

        @pl.when(ch0 + 1 < NCHUNKS)
        def _():
            chunk_body(ch0 + 1, dst1, src1, sem_e1)

        return 0

    lax.fori_loop(0, (NCHUNKS + 1) // 2, outer_body, 0)

    def fix_body(r, _):
        for c in range(D_FEAT // 16):
            sl = pl.ds(c * 16, 16)
            v = acc[r, sl]
            acc[r, sl] = jnp.where(v == _NEG_INF, 0.0, v)
        return 0

    lax.fori_loop(0, RANGE, fix_body, 0)

    def out_body(b, _):
        @pl.when(lo + b * 16 < N_NODES)
        def _():
            pltpu.sync_copy(acc.at[pl.ds(b * 16, 16)],
                            out_hbm.at[pl.ds(lo + b * 16, 16)])
        return 0

    lax.fori_loop(0, NBLOCKS_OUT, out_body, 0)


@jax.jit
def _seg_max(node_feats, src, dst):
    mesh = plsc.VectorSubcoreMesh(core_axis_name="c", subcore_axis_name="s")
    f = functools.partial(
        pl.kernel,
        mesh=mesh,
        out_type=jax.ShapeDtypeStruct((N_NODES, D_FEAT), jnp.float32),
        scratch_types=[
            pltpu.VMEM((CHUNK,), jnp.int32),
            pltpu.VMEM((CHUNK,), jnp.int32),
            pltpu.VMEM((CHUNK,), jnp.int32),
            pltpu.VMEM((CHUNK,), jnp.int32),
            pltpu.VMEM((CHUNK + 16,), jnp.int32),
            pltpu.VMEM((NRING, 16, D_FEAT), jnp.float32),
            pltpu.VMEM((RANGE + 1, D_FEAT), jnp.float32),
            pltpu.SemaphoreType.DMA,
            pltpu.SemaphoreType.DMA,
            pltpu.SemaphoreType.DMA,
        ],
        compiler_params=pltpu.CompilerParams(needs_layout_passes=False),
    )(_seg_max_kernel)
    return f(node_feats, src, dst)


def kernel(node_feats, edge_index):
    ei = edge_index.astype(jnp.int32)
    return _seg_max(node_feats, ei[0], ei[1])
